# baseline (speedup 1.0000x reference)
import functools

import jax
import jax.numpy as jnp
from jax import lax
from jax.experimental import pallas as pl
from jax.experimental.pallas import tpu as pltpu, tpu_sc as plsc

N = 100000
E = 1600000
NPAD = 100096
EPAD = 1638400
NC, NS, L = 2, 16, 16
ROWS_PER_SUB = NPAD // NS
K = 128
SUP = 4
SG = 5

_mesh = plsc.VectorSubcoreMesh(core_axis_name="c", subcore_axis_name="s")
_sc_params = pltpu.CompilerParams(use_tc_tiling_on_sc=False)


@functools.partial(
    pl.kernel,
    out_type=jax.ShapeDtypeStruct((NC, NPAD, 16), jnp.float32),
    mesh=_mesh,
    compiler_params=_sc_params,
    scratch_types=[
        pltpu.VMEM((2, SUP, K), jnp.int32),
        pltpu.VMEM((2, SUP, K), jnp.int32),
        pltpu.VMEM((2, SUP, K, 16), jnp.float32),
        pltpu.VMEM_SHARED((NPAD, 16), jnp.float32),
        pltpu.SemaphoreType.DMA,
        pltpu.SemaphoreType.DMA,
    ],
)
def _sc_deg_agg1(x16, srca, dsta, zeros16, out, sb, db, rows, acc, gsem, ssem):
    c = lax.axis_index("c")
    s = lax.axis_index("s")
    tid = c * NS + s
    r0 = s * ROWS_PER_SUB
    pltpu.sync_copy(zeros16.at[pl.ds(r0, ROWS_PER_SUB)], acc.at[pl.ds(r0, ROWS_PER_SUB)])
    plsc.subcore_barrier()
    nsup = EPAD // (NC * NS) // K // SUP

    def body(g, carry):
        slot = g % 2
        pltpu.sync_copy(srca.at[tid, pl.ds(g * SUP, SUP)], sb.at[slot])
        pltpu.sync_copy(dsta.at[tid, pl.ds(g * SUP, SUP)], db.at[slot])

        @pl.when(g >= 2)
        def _():
            for r in range(SUP):
                pltpu.make_async_copy(zeros16.at[pl.ds(0, K)], rows.at[0, r], ssem).wait()

        cps = [pltpu.async_copy(x16.at[sb.at[slot, r]], rows.at[slot, r], gsem)
               for r in range(SUP)]
        for cp in cps:
            cp.wait()
        for r in range(SUP):
            pltpu.async_copy(rows.at[slot, r], acc.at[db.at[slot, r]], ssem, add=True)
        return carry

    lax.fori_loop(0, nsup, body, 0)
    for r in range(2 * SUP):
        pltpu.make_async_copy(zeros16.at[pl.ds(0, K)], rows.at[0, r % SUP], ssem).wait()
    plsc.subcore_barrier()
    pltpu.sync_copy(acc.at[pl.ds(r0, ROWS_PER_SUB)], out.at[c, pl.ds(r0, ROWS_PER_SUB)])


@functools.partial(
    pl.kernel,
    out_type=jax.ShapeDtypeStruct((2, NC, NPAD, 16), jnp.float32),
    mesh=_mesh,
    compiler_params=_sc_params,
    scratch_types=[
        pltpu.VMEM((2, SUP, K), jnp.int32),
        pltpu.VMEM((2, SUP, K), jnp.int32),
        pltpu.VMEM((2, SUP, K, 16), jnp.float32),
        pltpu.VMEM_SHARED((NPAD, 16), jnp.float32),
        pltpu.SemaphoreType.DMA,
        pltpu.SemaphoreType.DMA,
    ],
)
def _sc_agg2(h1flat, srcb, dstb, zeros16, out, sb, db, rows, acc, gsem, ssem):
    c = lax.axis_index("c")
    s = lax.axis_index("s")
    r0 = s * ROWS_PER_SUB
    nsup = EPAD // NS // K // SUP
    for p in range(2):
        chunk = p * NC + c
        off = chunk * NPAD
        pltpu.sync_copy(zeros16.at[pl.ds(r0, ROWS_PER_SUB)],
                        acc.at[pl.ds(r0, ROWS_PER_SUB)])
        plsc.subcore_barrier()

        def body(g, carry):
            slot = g % 2
            pltpu.sync_copy(srcb.at[s, pl.ds(g * SUP, SUP)], sb.at[slot])
            pltpu.sync_copy(dstb.at[s, pl.ds(g * SUP, SUP)], db.at[slot])
            for r in range(SUP):
                for t in range(K // L):
                    sb[slot, r, pl.ds(t * L, L)] = sb[slot, r, pl.ds(t * L, L)] + off

            @pl.when(g >= 2)
            def _():
                for r in range(SUP):
                    pltpu.make_async_copy(zeros16.at[pl.ds(0, K)], rows.at[0, r],
                                          ssem).wait()

            cps = [pltpu.async_copy(h1flat.at[sb.at[slot, r]], rows.at[slot, r], gsem)
                   for r in range(SUP)]
            for cp in cps:
                cp.wait()
            for r in range(SUP):
                pltpu.async_copy(rows.at[slot, r], acc.at[db.at[slot, r]], ssem,
                                 add=True)
            return carry

        lax.fori_loop(0, nsup, body, 0)
        for r in range(2 * SUP):
            pltpu.make_async_copy(zeros16.at[pl.ds(0, K)], rows.at[0, r % SUP],
                                  ssem).wait()
        plsc.subcore_barrier()
        pltpu.sync_copy(acc.at[pl.ds(r0, ROWS_PER_SUB)],
                        out.at[p, c, pl.ds(r0, ROWS_PER_SUB)])
        plsc.subcore_barrier()


@functools.partial(
    pl.kernel,
    out_type=jax.ShapeDtypeStruct((EPAD, 128), jnp.bfloat16),
    mesh=_mesh,
    compiler_params=_sc_params,
    scratch_types=[
        pltpu.VMEM((2, SG, K), jnp.int32),
        pltpu.VMEM((2, SG * K, 64), jnp.bfloat16),
        pltpu.SemaphoreType.DMA,
        pltpu.SemaphoreType.DMA,
    ],
)
def _sc_gather_pair(h2, srca, dsta, gpair, ib, rows, gsem, wsem):
    c = lax.axis_index("c")
    s = lax.axis_index("s")
    tid = c * NS + s
    steps = EPAD // (NC * NS) // K
    nsup = steps // SG
    for idx3, col0 in ((srca, 0), (dsta, 64)):
        def body(g, carry, idx3=idx3, col0=col0):
            slot = g % 2
            pltpu.sync_copy(idx3.at[tid, pl.ds(g * SG, SG)], ib.at[slot])

            @pl.when(g >= 2)
            def _():
                pltpu.make_async_copy(gpair.at[pl.ds(0, SG * K), pl.ds(col0, 64)],
                                      rows.at[0], wsem).wait()

            cps = [pltpu.async_copy(h2.at[ib.at[slot, r]],
                                    rows.at[slot, pl.ds(r * K, K)], gsem)
                   for r in range(SG)]
            for cp in cps:
                cp.wait()
            pltpu.async_copy(
                rows.at[slot],
                gpair.at[pl.ds((tid * steps + g * SG) * K, SG * K),
                         pl.ds(col0, 64)], wsem)
            return carry

        lax.fori_loop(0, nsup, body, 0)
        for _ in range(2):
            pltpu.make_async_copy(gpair.at[pl.ds(0, SG * K), pl.ds(col0, 64)],
                                  rows.at[0], wsem).wait()


def _tc_h1_body(x4, parts, ws1, wn1, b1, h1c, dinv_o):
    agg = parts[0][:, :4] + parts[1][:, :4]
    dinv = 1.0 / jnp.maximum(agg[:, 3:4], 1.0)
    mean4 = agg * dinv
    h = jnp.dot(x4[...], ws1[...], preferred_element_type=jnp.float32)
    h += jnp.dot(mean4, wn1[...], preferred_element_type=jnp.float32)
    h = jnp.maximum(h + b1[...], 0.0)
    dinv_o[...] = dinv
    for cc in range(4):
        h1c[cc] = h[:, cc * 16:(cc + 1) * 16]


def _tc_h2_body(h1c, agg2, dinv, ws2, wn2, b2, h2_o):
    dv = dinv[...]
    h = b2[...] + jnp.zeros((h1c.shape[1], 64), jnp.float32)
    for chunk in range(4):
        p, c = chunk // NC, chunk % NC
        h += jnp.dot(h1c[chunk], ws2[pl.ds(chunk * 16, 16), :],
                     preferred_element_type=jnp.float32)
        h += jnp.dot(agg2[p, c] * dv, wn2[pl.ds(chunk * 16, 16), :],
                     preferred_element_type=jnp.float32)
    h2_o[...] = jnp.maximum(h, 0.0).astype(jnp.bfloat16)


def _tc_mlp_body(gp, ef, wa, wb, wef, be1, we2, be2, we3, be3, out):
    g32 = gp[...].astype(jnp.float32)
    z = jnp.dot(g32[:, :64], wa[...], preferred_element_type=jnp.float32)
    z += jnp.dot(g32[:, 64:], wb[...], preferred_element_type=jnp.float32)
    z += jnp.dot(ef[...], wef[...], preferred_element_type=jnp.float32)
    z = jnp.maximum(z + be1[...], 0.0)
    x = jnp.maximum(jnp.dot(z, we2[...], preferred_element_type=jnp.float32)
                    + be2[...], 0.0)
    out[...] = jnp.dot(x, we3[...], preferred_element_type=jnp.float32) + be3[...]


_NB = 256
_NG = NPAD // _NB
_EB = 512
_EG = EPAD // _EB


def _full(shape):
    return pl.BlockSpec(shape, lambda i: (0,) * len(shape))


def kernel(node_feats, edge_index, edge_feats, Ws1, Wn1, b1, Ws2, Wn2, b2,
           We1, be1, We2, be2, We3, be3):
    src = edge_index[0].astype(jnp.int32)
    dst = edge_index[1].astype(jnp.int32)
    src_p = jnp.concatenate([src, jnp.zeros((EPAD - E,), jnp.int32)])
    dst_p = jnp.concatenate([dst, jnp.full((EPAD - E,), N, jnp.int32)])
    srca = src_p.reshape(NC * NS, -1, K)
    dsta = dst_p.reshape(NC * NS, -1, K)
    srcb = src_p.reshape(NS, -1, K)
    dstb = dst_p.reshape(NS, -1, K)

    x4 = jnp.pad(jnp.concatenate(
        [node_feats, jnp.ones((N, 1), jnp.float32)], axis=1),
        ((0, NPAD - N), (0, 0)))
    x16 = jnp.pad(x4, ((0, 0), (0, 12)))
    zeros16 = jnp.zeros((NPAD, 16), jnp.float32)
    ws1p = jnp.pad(Ws1, ((0, 1), (0, 0)))
    wn1p = jnp.pad(Wn1, ((0, 1), (0, 0)))

    agg1 = _sc_deg_agg1(x16, srca, dsta, zeros16)

    h1c, dinv = pl.pallas_call(
        _tc_h1_body,
        grid=(_NG,),
        in_specs=[
            pl.BlockSpec((_NB, 4), lambda i: (i, 0)),
            pl.BlockSpec((NC, _NB, 16), lambda i: (0, i, 0)),
            _full((4, 64)), _full((4, 64)), _full((1, 64)),
        ],
        out_specs=[
            pl.BlockSpec((4, _NB, 16), lambda i: (0, i, 0)),
            pl.BlockSpec((_NB, 1), lambda i: (i, 0)),
        ],
        out_shape=[
            jax.ShapeDtypeStruct((4, NPAD, 16), jnp.float32),
            jax.ShapeDtypeStruct((NPAD, 1), jnp.float32),
        ],
    )(x4, agg1, ws1p, wn1p, b1.reshape(1, 64))

    agg2 = _sc_agg2(h1c.reshape(4 * NPAD, 16), srcb, dstb, zeros16)

    h2 = pl.pallas_call(
        _tc_h2_body,
        grid=(_NG,),
        in_specs=[
            pl.BlockSpec((4, _NB, 16), lambda i: (0, i, 0)),
            pl.BlockSpec((2, NC, _NB, 16), lambda i: (0, 0, i, 0)),
            pl.BlockSpec((_NB, 1), lambda i: (i, 0)),
            _full((64, 64)), _full((64, 64)), _full((1, 64)),
        ],
        out_specs=pl.BlockSpec((_NB, 64), lambda i: (i, 0)),
        out_shape=jax.ShapeDtypeStruct((NPAD, 64), jnp.bfloat16),
    )(h1c, agg2, dinv, Ws2, Wn2, b2.reshape(1, 64))

    gpair = _sc_gather_pair(h2, srca, dsta).reshape(EPAD, 128)

    efp = jnp.pad(edge_feats, ((0, EPAD - E), (0, 0)))
    logits = pl.pallas_call(
        _tc_mlp_body,
        grid=(_EG,),
        in_specs=[
            pl.BlockSpec((_EB, 128), lambda i: (i, 0)),
            pl.BlockSpec((_EB, 4), lambda i: (i, 0)),
            _full((64, 128)), _full((64, 128)), _full((4, 128)),
            _full((1, 128)), _full((128, 64)), _full((1, 64)),
            _full((64, 1)), _full((1, 1)),
        ],
        out_specs=pl.BlockSpec((_EB, 1), lambda i: (i, 0)),
        out_shape=jax.ShapeDtypeStruct((EPAD, 1), jnp.float32),
    )(gpair, efp, We1[:64], We1[64:128], We1[128:],
      be1.reshape(1, 128), We2, be2.reshape(1, 64), We3, be3.reshape(1, 1))

    return logits[:E]


# speedup vs baseline: 1.2351x; 1.2351x over previous
import functools

import jax
import jax.numpy as jnp
from jax import lax
from jax.experimental import pallas as pl
from jax.experimental.pallas import tpu as pltpu, tpu_sc as plsc

N = 100000
E = 1600000
NPAD = 100096
EPAD = 1638400
NC, NS, L = 2, 16, 16
ROWS_PER_SUB = NPAD // NS
K = 128
SUP = 4
SG = 5

_mesh = plsc.VectorSubcoreMesh(core_axis_name="c", subcore_axis_name="s")
_sc_params = pltpu.CompilerParams(use_tc_tiling_on_sc=False)


@functools.partial(
    pl.kernel,
    out_type=jax.ShapeDtypeStruct((NC, NPAD, 16), jnp.float32),
    mesh=_mesh,
    compiler_params=_sc_params,
    scratch_types=[
        pltpu.VMEM((2, SUP, K), jnp.int32),
        pltpu.VMEM((2, SUP, K), jnp.int32),
        pltpu.VMEM((2, SUP, K, 16), jnp.float32),
        pltpu.VMEM_SHARED((NPAD, 16), jnp.float32),
        pltpu.SemaphoreType.DMA,
        pltpu.SemaphoreType.DMA,
    ],
)
def _sc_deg_agg1(x16, srca, dsta, zeros16, out, sb, db, rows, acc, gsem, ssem):
    c = lax.axis_index("c")
    s = lax.axis_index("s")
    tid = c * NS + s
    r0 = s * ROWS_PER_SUB
    pltpu.sync_copy(zeros16.at[pl.ds(r0, ROWS_PER_SUB)], acc.at[pl.ds(r0, ROWS_PER_SUB)])
    plsc.subcore_barrier()
    nsup = EPAD // (NC * NS) // K // SUP

    def body(g, carry):
        slot = g % 2
        pltpu.sync_copy(srca.at[tid, pl.ds(g * SUP, SUP)], sb.at[slot])
        pltpu.sync_copy(dsta.at[tid, pl.ds(g * SUP, SUP)], db.at[slot])

        @pl.when(g >= 2)
        def _():
            for r in range(SUP):
                pltpu.make_async_copy(zeros16.at[pl.ds(0, K)], rows.at[0, r], ssem).wait()

        cps = [pltpu.async_copy(x16.at[sb.at[slot, r]], rows.at[slot, r], gsem)
               for r in range(SUP)]
        for cp in cps:
            cp.wait()
        for r in range(SUP):
            pltpu.async_copy(rows.at[slot, r], acc.at[db.at[slot, r]], ssem, add=True)
        return carry

    lax.fori_loop(0, nsup, body, 0)
    for r in range(2 * SUP):
        pltpu.make_async_copy(zeros16.at[pl.ds(0, K)], rows.at[0, r % SUP], ssem).wait()
    plsc.subcore_barrier()
    pltpu.sync_copy(acc.at[pl.ds(r0, ROWS_PER_SUB)], out.at[c, pl.ds(r0, ROWS_PER_SUB)])


@functools.partial(
    pl.kernel,
    out_type=jax.ShapeDtypeStruct((2, NC, NPAD, 16), jnp.float32),
    mesh=_mesh,
    compiler_params=_sc_params,
    scratch_types=[
        pltpu.VMEM((2, SUP, K), jnp.int32),
        pltpu.VMEM((2, SUP, K), jnp.int32),
        pltpu.VMEM((2, SUP, K, 16), jnp.float32),
        pltpu.VMEM_SHARED((NPAD, 16), jnp.float32),
        pltpu.SemaphoreType.DMA,
        pltpu.SemaphoreType.DMA,
    ],
)
def _sc_agg2(h1flat, srcb, dstb, zeros16, out, sb, db, rows, acc, gsem, ssem):
    c = lax.axis_index("c")
    s = lax.axis_index("s")
    r0 = s * ROWS_PER_SUB
    nsup = EPAD // NS // K // SUP
    for p in range(2):
        chunk = p * NC + c
        off = chunk * NPAD
        pltpu.sync_copy(zeros16.at[pl.ds(r0, ROWS_PER_SUB)],
                        acc.at[pl.ds(r0, ROWS_PER_SUB)])
        plsc.subcore_barrier()

        def body(g, carry):
            slot = g % 2
            pltpu.sync_copy(srcb.at[s, pl.ds(g * SUP, SUP)], sb.at[slot])
            pltpu.sync_copy(dstb.at[s, pl.ds(g * SUP, SUP)], db.at[slot])
            for r in range(SUP):
                for t in range(K // L):
                    sb[slot, r, pl.ds(t * L, L)] = sb[slot, r, pl.ds(t * L, L)] + off

            @pl.when(g >= 2)
            def _():
                for r in range(SUP):
                    pltpu.make_async_copy(zeros16.at[pl.ds(0, K)], rows.at[0, r],
                                          ssem).wait()

            cps = [pltpu.async_copy(h1flat.at[sb.at[slot, r]], rows.at[slot, r], gsem)
                   for r in range(SUP)]
            for cp in cps:
                cp.wait()
            for r in range(SUP):
                pltpu.async_copy(rows.at[slot, r], acc.at[db.at[slot, r]], ssem,
                                 add=True)
            return carry

        lax.fori_loop(0, nsup, body, 0)
        for r in range(2 * SUP):
            pltpu.make_async_copy(zeros16.at[pl.ds(0, K)], rows.at[0, r % SUP],
                                  ssem).wait()
        plsc.subcore_barrier()
        pltpu.sync_copy(acc.at[pl.ds(r0, ROWS_PER_SUB)],
                        out.at[p, c, pl.ds(r0, ROWS_PER_SUB)])
        plsc.subcore_barrier()


@functools.partial(
    pl.kernel,
    out_type=jax.ShapeDtypeStruct((EPAD, 128), jnp.bfloat16),
    mesh=_mesh,
    compiler_params=_sc_params,
    scratch_types=[
        pltpu.VMEM((2, SG, K), jnp.int32),
        pltpu.VMEM((2, SG * K, 64), jnp.bfloat16),
        pltpu.SemaphoreType.DMA,
        pltpu.SemaphoreType.DMA,
    ],
)
def _sc_gather_pair(h2, srca, dsta, gpair, ib, rows, gsem, wsem):
    c = lax.axis_index("c")
    s = lax.axis_index("s")
    tid = c * NS + s
    steps = EPAD // (NC * NS) // K
    nsup = steps // SG
    for idx3, col0 in ((srca, 0), (dsta, 64)):
        def body(g, carry, idx3=idx3, col0=col0):
            slot = g % 2
            pltpu.sync_copy(idx3.at[tid, pl.ds(g * SG, SG)], ib.at[slot])

            @pl.when(g >= 2)
            def _():
                pltpu.make_async_copy(gpair.at[pl.ds(0, SG * K), pl.ds(col0, 64)],
                                      rows.at[0], wsem).wait()

            cps = [pltpu.async_copy(h2.at[ib.at[slot, r]],
                                    rows.at[slot, pl.ds(r * K, K)], gsem)
                   for r in range(SG)]
            for cp in cps:
                cp.wait()
            pltpu.async_copy(
                rows.at[slot],
                gpair.at[pl.ds((tid * steps + g * SG) * K, SG * K),
                         pl.ds(col0, 64)], wsem)
            return carry

        lax.fori_loop(0, nsup, body, 0)
        for _ in range(2):
            pltpu.make_async_copy(gpair.at[pl.ds(0, SG * K), pl.ds(col0, 64)],
                                  rows.at[0], wsem).wait()


def _tc_h1_body(x4, parts, ws1, wn1, b1, h1c, dinv_o):
    agg = parts[0][:, :4] + parts[1][:, :4]
    dinv = 1.0 / jnp.maximum(agg[:, 3:4], 1.0)
    mean4 = agg * dinv
    h = jnp.dot(x4[...], ws1[...], preferred_element_type=jnp.float32)
    h += jnp.dot(mean4, wn1[...], preferred_element_type=jnp.float32)
    h = jnp.maximum(h + b1[...], 0.0)
    dinv_o[...] = dinv
    for cc in range(4):
        h1c[cc] = h[:, cc * 16:(cc + 1) * 16]


def _tc_h2_body(h1c, agg2, dinv, ws2, wn2, b2, h2_o):
    dv = dinv[...]
    h = b2[...] + jnp.zeros((h1c.shape[1], 64), jnp.float32)
    for chunk in range(4):
        p, c = chunk // NC, chunk % NC
        h += jnp.dot(h1c[chunk], ws2[pl.ds(chunk * 16, 16), :],
                     preferred_element_type=jnp.float32)
        h += jnp.dot(agg2[p, c] * dv, wn2[pl.ds(chunk * 16, 16), :],
                     preferred_element_type=jnp.float32)
    h2_o[...] = jnp.maximum(h, 0.0).astype(jnp.bfloat16)


def _tc_mlp_body(gp, ef, wa, wb, wef, be1, we2, be2, we3, be3, out):
    g32 = gp[...].astype(jnp.float32)
    z = jnp.dot(g32[:, :64], wa[...], preferred_element_type=jnp.float32)
    z += jnp.dot(g32[:, 64:], wb[...], preferred_element_type=jnp.float32)
    z += jnp.dot(ef[...], wef[...], preferred_element_type=jnp.float32)
    z = jnp.maximum(z + be1[...], 0.0)
    x = jnp.maximum(jnp.dot(z, we2[...], preferred_element_type=jnp.float32)
                    + be2[...], 0.0)
    out[...] = jnp.dot(x, we3[...], preferred_element_type=jnp.float32) + be3[...]


_NB = 256
_NG = NPAD // _NB
_EB = 512
_EG = E // _EB


def _full(shape):
    return pl.BlockSpec(shape, lambda i: (0,) * len(shape))


def kernel(node_feats, edge_index, edge_feats, Ws1, Wn1, b1, Ws2, Wn2, b2,
           We1, be1, We2, be2, We3, be3):
    src = edge_index[0].astype(jnp.int32)
    dst = edge_index[1].astype(jnp.int32)
    src_p = jnp.concatenate([src, jnp.zeros((EPAD - E,), jnp.int32)])
    dst_p = jnp.concatenate([dst, jnp.full((EPAD - E,), N, jnp.int32)])
    srca = src_p.reshape(NC * NS, -1, K)
    dsta = dst_p.reshape(NC * NS, -1, K)
    srcb = src_p.reshape(NS, -1, K)
    dstb = dst_p.reshape(NS, -1, K)

    x4 = jnp.pad(jnp.concatenate(
        [node_feats, jnp.ones((N, 1), jnp.float32)], axis=1),
        ((0, NPAD - N), (0, 0)))
    x16 = jnp.pad(x4, ((0, 0), (0, 12)))
    zeros16 = jnp.zeros((NPAD, 16), jnp.float32)
    ws1p = jnp.pad(Ws1, ((0, 1), (0, 0)))
    wn1p = jnp.pad(Wn1, ((0, 1), (0, 0)))

    agg1 = _sc_deg_agg1(x16, srca, dsta, zeros16)

    h1c, dinv = pl.pallas_call(
        _tc_h1_body,
        grid=(_NG,),
        in_specs=[
            pl.BlockSpec((_NB, 4), lambda i: (i, 0)),
            pl.BlockSpec((NC, _NB, 16), lambda i: (0, i, 0)),
            _full((4, 64)), _full((4, 64)), _full((1, 64)),
        ],
        out_specs=[
            pl.BlockSpec((4, _NB, 16), lambda i: (0, i, 0)),
            pl.BlockSpec((_NB, 1), lambda i: (i, 0)),
        ],
        out_shape=[
            jax.ShapeDtypeStruct((4, NPAD, 16), jnp.float32),
            jax.ShapeDtypeStruct((NPAD, 1), jnp.float32),
        ],
    )(x4, agg1, ws1p, wn1p, b1.reshape(1, 64))

    agg2 = _sc_agg2(h1c.reshape(4 * NPAD, 16), srcb, dstb, zeros16)

    h2 = pl.pallas_call(
        _tc_h2_body,
        grid=(_NG,),
        in_specs=[
            pl.BlockSpec((4, _NB, 16), lambda i: (0, i, 0)),
            pl.BlockSpec((2, NC, _NB, 16), lambda i: (0, 0, i, 0)),
            pl.BlockSpec((_NB, 1), lambda i: (i, 0)),
            _full((64, 64)), _full((64, 64)), _full((1, 64)),
        ],
        out_specs=pl.BlockSpec((_NB, 64), lambda i: (i, 0)),
        out_shape=jax.ShapeDtypeStruct((NPAD, 64), jnp.bfloat16),
    )(h1c, agg2, dinv, Ws2, Wn2, b2.reshape(1, 64))

    gpair = _sc_gather_pair(h2, srca, dsta).reshape(EPAD, 128)

    logits = pl.pallas_call(
        _tc_mlp_body,
        grid=(_EG,),
        in_specs=[
            pl.BlockSpec((_EB, 128), lambda i: (i, 0)),
            pl.BlockSpec((_EB, 4), lambda i: (i, 0)),
            _full((64, 128)), _full((64, 128)), _full((4, 128)),
            _full((1, 128)), _full((128, 64)), _full((1, 64)),
            _full((64, 1)), _full((1, 1)),
        ],
        out_specs=pl.BlockSpec((_EB, 1), lambda i: (i, 0)),
        out_shape=jax.ShapeDtypeStruct((E, 1), jnp.float32),
    )(gpair, edge_feats, We1[:64], We1[64:128], We1[128:],
      be1.reshape(1, 128), We2, be2.reshape(1, 64), We3, be3.reshape(1, 1))

    return logits


# speedup vs baseline: 1.5250x; 1.2348x over previous
import functools

import jax
import jax.numpy as jnp
from jax import lax
from jax.experimental import pallas as pl
from jax.experimental.pallas import tpu as pltpu, tpu_sc as plsc

N = 100000
E = 1600000
NPAD = 100096
EPAD = 1638400
NC, NS, L = 2, 16, 16
ROWS_PER_SUB = NPAD // NS
K = 128
SUP = 4
SG = 5

_mesh = plsc.VectorSubcoreMesh(core_axis_name="c", subcore_axis_name="s")
_sc_params = pltpu.CompilerParams(use_tc_tiling_on_sc=False)


@functools.partial(
    pl.kernel,
    out_type=jax.ShapeDtypeStruct((NC, NPAD, 16), jnp.float32),
    mesh=_mesh,
    compiler_params=_sc_params,
    scratch_types=[
        pltpu.VMEM((2, SUP, K), jnp.int32),
        pltpu.VMEM((2, SUP, K), jnp.int32),
        pltpu.VMEM((2, SUP, K, 16), jnp.float32),
        pltpu.VMEM_SHARED((NPAD, 16), jnp.float32),
        pltpu.SemaphoreType.DMA,
        pltpu.SemaphoreType.DMA,
    ],
)
def _sc_deg_agg1(x16, srca, dsta, zeros16, out, sb, db, rows, acc, gsem, ssem):
    c = lax.axis_index("c")
    s = lax.axis_index("s")
    tid = c * NS + s
    r0 = s * ROWS_PER_SUB
    pltpu.sync_copy(zeros16.at[pl.ds(r0, ROWS_PER_SUB)], acc.at[pl.ds(r0, ROWS_PER_SUB)])
    plsc.subcore_barrier()
    nsup = EPAD // (NC * NS) // K // SUP

    def body(g, carry):
        slot = g % 2
        pltpu.sync_copy(srca.at[tid, pl.ds(g * SUP, SUP)], sb.at[slot])
        pltpu.sync_copy(dsta.at[tid, pl.ds(g * SUP, SUP)], db.at[slot])

        @pl.when(g >= 2)
        def _():
            for r in range(SUP):
                pltpu.make_async_copy(zeros16.at[pl.ds(0, K)], rows.at[0, r], ssem).wait()

        cps = [pltpu.async_copy(x16.at[sb.at[slot, r]], rows.at[slot, r], gsem)
               for r in range(SUP)]
        for cp in cps:
            cp.wait()
        for r in range(SUP):
            pltpu.async_copy(rows.at[slot, r], acc.at[db.at[slot, r]], ssem, add=True)
        return carry

    lax.fori_loop(0, nsup, body, 0)
    for r in range(2 * SUP):
        pltpu.make_async_copy(zeros16.at[pl.ds(0, K)], rows.at[0, r % SUP], ssem).wait()
    plsc.subcore_barrier()
    pltpu.sync_copy(acc.at[pl.ds(r0, ROWS_PER_SUB)], out.at[c, pl.ds(r0, ROWS_PER_SUB)])


@functools.partial(
    pl.kernel,
    out_type=jax.ShapeDtypeStruct((2, NC, NPAD, 16), jnp.float32),
    mesh=_mesh,
    compiler_params=_sc_params,
    scratch_types=[
        pltpu.VMEM((2, SUP, K), jnp.int32),
        pltpu.VMEM((2, SUP, K), jnp.int32),
        pltpu.VMEM((2, SUP, K, 16), jnp.float32),
        pltpu.VMEM_SHARED((NPAD, 16), jnp.float32),
        pltpu.SemaphoreType.DMA,
        pltpu.SemaphoreType.DMA,
    ],
)
def _sc_agg2(h1flat, srcb, dstb, zeros16, out, sb, db, rows, acc, gsem, ssem):
    c = lax.axis_index("c")
    s = lax.axis_index("s")
    r0 = s * ROWS_PER_SUB
    nsup = EPAD // NS // K // SUP
    for p in range(2):
        chunk = p * NC + c
        off = chunk * NPAD
        pltpu.sync_copy(zeros16.at[pl.ds(r0, ROWS_PER_SUB)],
                        acc.at[pl.ds(r0, ROWS_PER_SUB)])
        plsc.subcore_barrier()

        def body(g, carry):
            slot = g % 2
            pltpu.sync_copy(srcb.at[s, pl.ds(g * SUP, SUP)], sb.at[slot])
            pltpu.sync_copy(dstb.at[s, pl.ds(g * SUP, SUP)], db.at[slot])
            for r in range(SUP):
                for t in range(K // L):
                    sb[slot, r, pl.ds(t * L, L)] = sb[slot, r, pl.ds(t * L, L)] + off

            @pl.when(g >= 2)
            def _():
                for r in range(SUP):
                    pltpu.make_async_copy(zeros16.at[pl.ds(0, K)], rows.at[0, r],
                                          ssem).wait()

            cps = [pltpu.async_copy(h1flat.at[sb.at[slot, r]], rows.at[slot, r], gsem)
                   for r in range(SUP)]
            for cp in cps:
                cp.wait()
            for r in range(SUP):
                pltpu.async_copy(rows.at[slot, r], acc.at[db.at[slot, r]], ssem,
                                 add=True)
            return carry

        lax.fori_loop(0, nsup, body, 0)
        for r in range(2 * SUP):
            pltpu.make_async_copy(zeros16.at[pl.ds(0, K)], rows.at[0, r % SUP],
                                  ssem).wait()
        plsc.subcore_barrier()
        pltpu.sync_copy(acc.at[pl.ds(r0, ROWS_PER_SUB)],
                        out.at[p, c, pl.ds(r0, ROWS_PER_SUB)])
        plsc.subcore_barrier()


@functools.partial(
    pl.kernel,
    out_type=jax.ShapeDtypeStruct((EPAD, 128), jnp.bfloat16),
    mesh=_mesh,
    compiler_params=_sc_params,
    scratch_types=[
        pltpu.VMEM((2, SG, K), jnp.int32),
        pltpu.VMEM((2, SG * K, 64), jnp.bfloat16),
        pltpu.SemaphoreType.DMA,
        pltpu.SemaphoreType.DMA,
    ],
)
def _sc_gather_pair(h2, srca, dsta, gpair, ib, rows, gsem, wsem):
    c = lax.axis_index("c")
    s = lax.axis_index("s")
    tid = c * NS + s
    steps = EPAD // (NC * NS) // K
    nsup = steps // SG
    for idx3, col0 in ((srca, 0), (dsta, 64)):
        def body(g, carry, idx3=idx3, col0=col0):
            slot = g % 2
            pltpu.sync_copy(idx3.at[tid, pl.ds(g * SG, SG)], ib.at[slot])

            @pl.when(g >= 2)
            def _():
                pltpu.make_async_copy(gpair.at[pl.ds(0, SG * K), pl.ds(col0, 64)],
                                      rows.at[0], wsem).wait()

            cps = [pltpu.async_copy(h2.at[ib.at[slot, r]],
                                    rows.at[slot, pl.ds(r * K, K)], gsem)
                   for r in range(SG)]
            for cp in cps:
                cp.wait()
            pltpu.async_copy(
                rows.at[slot],
                gpair.at[pl.ds((tid * steps + g * SG) * K, SG * K),
                         pl.ds(col0, 64)], wsem)
            return carry

        lax.fori_loop(0, nsup, body, 0)
        for _ in range(2):
            pltpu.make_async_copy(gpair.at[pl.ds(0, SG * K), pl.ds(col0, 64)],
                                  rows.at[0], wsem).wait()


def _tc_h1_body(x4, parts, ws1, wn1, b1, h1c, dinv_o):
    agg = parts[0][:, :4] + parts[1][:, :4]
    dinv = 1.0 / jnp.maximum(agg[:, 3:4], 1.0)
    mean4 = agg * dinv
    h = jnp.dot(x4[...], ws1[...], preferred_element_type=jnp.float32)
    h += jnp.dot(mean4, wn1[...], preferred_element_type=jnp.float32)
    h = jnp.maximum(h + b1[...], 0.0)
    dinv_o[...] = dinv
    for cc in range(4):
        h1c[cc] = h[:, cc * 16:(cc + 1) * 16]


def _tc_h2_body(h1c, agg2, dinv, ws2, wn2, b2, h2_o):
    dv = dinv[...]
    h = b2[...] + jnp.zeros((h1c.shape[1], 64), jnp.float32)
    for chunk in range(4):
        p, c = chunk // NC, chunk % NC
        h += jnp.dot(h1c[chunk], ws2[pl.ds(chunk * 16, 16), :],
                     preferred_element_type=jnp.float32)
        h += jnp.dot(agg2[p, c] * dv, wn2[pl.ds(chunk * 16, 16), :],
                     preferred_element_type=jnp.float32)
    h2_o[...] = jnp.maximum(h, 0.0).astype(jnp.bfloat16)


def _tc_mlp_body(gp, ef, wa, wb, wef, be1, we2, be2, we3, be3, out):
    g32 = gp[...].astype(jnp.float32)
    z = jnp.dot(g32[:, :64], wa[...], preferred_element_type=jnp.float32)
    z += jnp.dot(g32[:, 64:], wb[...], preferred_element_type=jnp.float32)
    z += jnp.dot(ef[...], wef[...], preferred_element_type=jnp.float32)
    z = jnp.maximum(z + be1[...], 0.0)
    x = jnp.maximum(jnp.dot(z, we2[...], preferred_element_type=jnp.float32)
                    + be2[...], 0.0)
    out[...] = jnp.dot(x, we3[...], preferred_element_type=jnp.float32) + be3[...]


_NB = 256
_NG = NPAD // _NB
_EB = 4000
_EG = E // _EB


def _full(shape):
    return pl.BlockSpec(shape, lambda i: (0,) * len(shape))


def kernel(node_feats, edge_index, edge_feats, Ws1, Wn1, b1, Ws2, Wn2, b2,
           We1, be1, We2, be2, We3, be3):
    src = edge_index[0].astype(jnp.int32)
    dst = edge_index[1].astype(jnp.int32)
    src_p = jnp.concatenate([src, jnp.zeros((EPAD - E,), jnp.int32)])
    dst_p = jnp.concatenate([dst, jnp.full((EPAD - E,), N, jnp.int32)])
    srca = src_p.reshape(NC * NS, -1, K)
    dsta = dst_p.reshape(NC * NS, -1, K)
    srcb = src_p.reshape(NS, -1, K)
    dstb = dst_p.reshape(NS, -1, K)

    x4 = jnp.pad(jnp.concatenate(
        [node_feats, jnp.ones((N, 1), jnp.float32)], axis=1),
        ((0, NPAD - N), (0, 0)))
    x16 = jnp.pad(x4, ((0, 0), (0, 12)))
    zeros16 = jnp.zeros((NPAD, 16), jnp.float32)
    ws1p = jnp.pad(Ws1, ((0, 1), (0, 0)))
    wn1p = jnp.pad(Wn1, ((0, 1), (0, 0)))

    agg1 = _sc_deg_agg1(x16, srca, dsta, zeros16)

    h1c, dinv = pl.pallas_call(
        _tc_h1_body,
        grid=(_NG,),
        in_specs=[
            pl.BlockSpec((_NB, 4), lambda i: (i, 0)),
            pl.BlockSpec((NC, _NB, 16), lambda i: (0, i, 0)),
            _full((4, 64)), _full((4, 64)), _full((1, 64)),
        ],
        out_specs=[
            pl.BlockSpec((4, _NB, 16), lambda i: (0, i, 0)),
            pl.BlockSpec((_NB, 1), lambda i: (i, 0)),
        ],
        out_shape=[
            jax.ShapeDtypeStruct((4, NPAD, 16), jnp.float32),
            jax.ShapeDtypeStruct((NPAD, 1), jnp.float32),
        ],
    )(x4, agg1, ws1p, wn1p, b1.reshape(1, 64))

    agg2 = _sc_agg2(h1c.reshape(4 * NPAD, 16), srcb, dstb, zeros16)

    h2 = pl.pallas_call(
        _tc_h2_body,
        grid=(_NG,),
        in_specs=[
            pl.BlockSpec((4, _NB, 16), lambda i: (0, i, 0)),
            pl.BlockSpec((2, NC, _NB, 16), lambda i: (0, 0, i, 0)),
            pl.BlockSpec((_NB, 1), lambda i: (i, 0)),
            _full((64, 64)), _full((64, 64)), _full((1, 64)),
        ],
        out_specs=pl.BlockSpec((_NB, 64), lambda i: (i, 0)),
        out_shape=jax.ShapeDtypeStruct((NPAD, 64), jnp.bfloat16),
    )(h1c, agg2, dinv, Ws2, Wn2, b2.reshape(1, 64))

    gpair = _sc_gather_pair(h2, srca, dsta).reshape(EPAD, 128)

    logits = pl.pallas_call(
        _tc_mlp_body,
        grid=(_EG,),
        in_specs=[
            pl.BlockSpec((_EB, 128), lambda i: (i, 0)),
            pl.BlockSpec((_EB, 4), lambda i: (i, 0)),
            _full((64, 128)), _full((64, 128)), _full((4, 128)),
            _full((1, 128)), _full((128, 64)), _full((1, 64)),
            _full((64, 1)), _full((1, 1)),
        ],
        out_specs=pl.BlockSpec((_EB, 1), lambda i: (i, 0)),
        out_shape=jax.ShapeDtypeStruct((E, 1), jnp.float32),
    )(gpair, edge_feats, We1[:64], We1[64:128], We1[128:],
      be1.reshape(1, 128), We2, be2.reshape(1, 64), We3, be3.reshape(1, 1))

    return logits


# speedup vs baseline: 1.6053x; 1.0526x over previous
import functools

import jax
import jax.numpy as jnp
from jax import lax
from jax.experimental import pallas as pl
from jax.experimental.pallas import tpu as pltpu, tpu_sc as plsc

N = 100000
E = 1600000
NPAD = 100096
EPAD = 1638400
NC, NS, L = 2, 16, 16
ROWS_PER_SUB = NPAD // NS
K = 128
SUP = 4
SG = 5

_mesh = plsc.VectorSubcoreMesh(core_axis_name="c", subcore_axis_name="s")
_sc_params = pltpu.CompilerParams(use_tc_tiling_on_sc=False)


@functools.partial(
    pl.kernel,
    out_type=jax.ShapeDtypeStruct((NC, NPAD, 16), jnp.float32),
    mesh=_mesh,
    compiler_params=_sc_params,
    scratch_types=[
        pltpu.VMEM((2, SUP, K), jnp.int32),
        pltpu.VMEM((2, SUP, K), jnp.int32),
        pltpu.VMEM((2, SUP, K, 16), jnp.float32),
        pltpu.VMEM_SHARED((NPAD, 16), jnp.float32),
        pltpu.SemaphoreType.DMA,
        pltpu.SemaphoreType.DMA,
    ],
)
def _sc_deg_agg1(x16, srca, dsta, zeros16, out, sb, db, rows, acc, gsem, ssem):
    c = lax.axis_index("c")
    s = lax.axis_index("s")
    tid = c * NS + s
    r0 = s * ROWS_PER_SUB
    pltpu.sync_copy(zeros16.at[pl.ds(r0, ROWS_PER_SUB)], acc.at[pl.ds(r0, ROWS_PER_SUB)])
    plsc.subcore_barrier()
    nsup = EPAD // (NC * NS) // K // SUP

    def body(g, carry):
        slot = g % 2
        pltpu.sync_copy(srca.at[tid, pl.ds(g * SUP, SUP)], sb.at[slot])
        pltpu.sync_copy(dsta.at[tid, pl.ds(g * SUP, SUP)], db.at[slot])

        @pl.when(g >= 2)
        def _():
            for r in range(SUP):
                pltpu.make_async_copy(zeros16.at[pl.ds(0, K)], rows.at[0, r], ssem).wait()

        cps = [pltpu.async_copy(x16.at[sb.at[slot, r]], rows.at[slot, r], gsem)
               for r in range(SUP)]
        for cp in cps:
            cp.wait()
        for r in range(SUP):
            pltpu.async_copy(rows.at[slot, r], acc.at[db.at[slot, r]], ssem, add=True)
        return carry

    lax.fori_loop(0, nsup, body, 0)
    for r in range(2 * SUP):
        pltpu.make_async_copy(zeros16.at[pl.ds(0, K)], rows.at[0, r % SUP], ssem).wait()
    plsc.subcore_barrier()
    pltpu.sync_copy(acc.at[pl.ds(r0, ROWS_PER_SUB)], out.at[c, pl.ds(r0, ROWS_PER_SUB)])


@functools.partial(
    pl.kernel,
    out_type=jax.ShapeDtypeStruct((2, NC, NPAD, 16), jnp.float32),
    mesh=_mesh,
    compiler_params=_sc_params,
    scratch_types=[
        pltpu.VMEM((2, SUP, K), jnp.int32),
        pltpu.VMEM((2, SUP, K), jnp.int32),
        pltpu.VMEM((2, SUP, K, 16), jnp.float32),
        pltpu.VMEM_SHARED((NPAD, 16), jnp.float32),
        pltpu.SemaphoreType.DMA,
        pltpu.SemaphoreType.DMA,
    ],
)
def _sc_agg2(h1flat, srcb, dstb, zeros16, out, sb, db, rows, acc, gsem, ssem):
    c = lax.axis_index("c")
    s = lax.axis_index("s")
    r0 = s * ROWS_PER_SUB
    nsup = EPAD // NS // K // SUP
    for p in range(2):
        chunk = p * NC + c
        off = chunk * NPAD
        pltpu.sync_copy(zeros16.at[pl.ds(r0, ROWS_PER_SUB)],
                        acc.at[pl.ds(r0, ROWS_PER_SUB)])
        plsc.subcore_barrier()

        def body(g, carry):
            slot = g % 2
            pltpu.sync_copy(srcb.at[s, pl.ds(g * SUP, SUP)], sb.at[slot])
            pltpu.sync_copy(dstb.at[s, pl.ds(g * SUP, SUP)], db.at[slot])
            for r in range(SUP):
                for t in range(K // L):
                    sb[slot, r, pl.ds(t * L, L)] = sb[slot, r, pl.ds(t * L, L)] + off

            @pl.when(g >= 2)
            def _():
                for r in range(SUP):
                    pltpu.make_async_copy(zeros16.at[pl.ds(0, K)], rows.at[0, r],
                                          ssem).wait()

            cps = [pltpu.async_copy(h1flat.at[sb.at[slot, r]], rows.at[slot, r], gsem)
                   for r in range(SUP)]
            for cp in cps:
                cp.wait()
            for r in range(SUP):
                pltpu.async_copy(rows.at[slot, r], acc.at[db.at[slot, r]], ssem,
                                 add=True)
            return carry

        lax.fori_loop(0, nsup, body, 0)
        for r in range(2 * SUP):
            pltpu.make_async_copy(zeros16.at[pl.ds(0, K)], rows.at[0, r % SUP],
                                  ssem).wait()
        plsc.subcore_barrier()
        pltpu.sync_copy(acc.at[pl.ds(r0, ROWS_PER_SUB)],
                        out.at[p, c, pl.ds(r0, ROWS_PER_SUB)])
        plsc.subcore_barrier()


@functools.partial(
    pl.kernel,
    out_type=jax.ShapeDtypeStruct((EPAD, 128), jnp.bfloat16),
    mesh=_mesh,
    compiler_params=_sc_params,
    scratch_types=[
        pltpu.VMEM((2, SG, K), jnp.int32),
        pltpu.VMEM((2, SG * K, 64), jnp.bfloat16),
        pltpu.SemaphoreType.DMA,
        pltpu.SemaphoreType.DMA,
    ],
)
def _sc_gather_pair(h2, srca, dsta, gpair, ib, rows, gsem, wsem):
    c = lax.axis_index("c")
    s = lax.axis_index("s")
    tid = c * NS + s
    steps = EPAD // (NC * NS) // K
    nsup = steps // SG
    for idx3, col0 in ((srca, 0), (dsta, 64)):
        def body(g, carry, idx3=idx3, col0=col0):
            slot = g % 2
            pltpu.sync_copy(idx3.at[tid, pl.ds(g * SG, SG)], ib.at[slot])

            @pl.when(g >= 2)
            def _():
                pltpu.make_async_copy(gpair.at[pl.ds(0, SG * K), pl.ds(col0, 64)],
                                      rows.at[0], wsem).wait()

            cps = [pltpu.async_copy(h2.at[ib.at[slot, r]],
                                    rows.at[slot, pl.ds(r * K, K)], gsem)
                   for r in range(SG)]
            for cp in cps:
                cp.wait()
            pltpu.async_copy(
                rows.at[slot],
                gpair.at[pl.ds((tid * steps + g * SG) * K, SG * K),
                         pl.ds(col0, 64)], wsem)
            return carry

        lax.fori_loop(0, nsup, body, 0)
        for _ in range(2):
            pltpu.make_async_copy(gpair.at[pl.ds(0, SG * K), pl.ds(col0, 64)],
                                  rows.at[0], wsem).wait()


def _tc_h1_body(x4, parts, ws1, wn1, b1, h1c, dinv_o):
    agg = parts[0][:, :4] + parts[1][:, :4]
    dinv = 1.0 / jnp.maximum(agg[:, 3:4], 1.0)
    mean4 = agg * dinv
    h = jnp.dot(x4[...], ws1[...], preferred_element_type=jnp.float32)
    h += jnp.dot(mean4, wn1[...], preferred_element_type=jnp.float32)
    h = jnp.maximum(h + b1[...], 0.0)
    dinv_o[...] = dinv
    for cc in range(4):
        h1c[cc] = h[:, cc * 16:(cc + 1) * 16]


def _tc_h2_body(h1c, agg2, dinv, ws2, wn2, b2, h2_o):
    dv = dinv[...]
    h = b2[...] + jnp.zeros((h1c.shape[1], 64), jnp.float32)
    for chunk in range(4):
        p, c = chunk // NC, chunk % NC
        h += jnp.dot(h1c[chunk], ws2[pl.ds(chunk * 16, 16), :],
                     preferred_element_type=jnp.float32)
        h += jnp.dot(agg2[p, c] * dv, wn2[pl.ds(chunk * 16, 16), :],
                     preferred_element_type=jnp.float32)
    h2_o[...] = jnp.maximum(h, 0.0).astype(jnp.bfloat16)


def _tc_mlp_body(gp, ef, wa, wb, wef, be1, we2, be2, we3, be3, out):
    g32 = gp[...].astype(jnp.float32)
    z = jnp.dot(g32[:, :64], wa[...], preferred_element_type=jnp.float32)
    z += jnp.dot(g32[:, 64:], wb[...], preferred_element_type=jnp.float32)
    z += jnp.dot(ef[...], wef[...], preferred_element_type=jnp.float32)
    z = jnp.maximum(z + be1[...], 0.0)
    x = jnp.maximum(jnp.dot(z, we2[...], preferred_element_type=jnp.float32)
                    + be2[...], 0.0)
    out[...] = jnp.dot(x, we3[...], preferred_element_type=jnp.float32) + be3[...]


_NB = 4352
_NG = NPAD // _NB
_EB = 4000
_EG = E // _EB


def _full(shape):
    return pl.BlockSpec(shape, lambda i: (0,) * len(shape))


def kernel(node_feats, edge_index, edge_feats, Ws1, Wn1, b1, Ws2, Wn2, b2,
           We1, be1, We2, be2, We3, be3):
    src = edge_index[0].astype(jnp.int32)
    dst = edge_index[1].astype(jnp.int32)
    src_p = jnp.concatenate([src, jnp.zeros((EPAD - E,), jnp.int32)])
    dst_p = jnp.concatenate([dst, jnp.full((EPAD - E,), N, jnp.int32)])
    srca = src_p.reshape(NC * NS, -1, K)
    dsta = dst_p.reshape(NC * NS, -1, K)
    srcb = src_p.reshape(NS, -1, K)
    dstb = dst_p.reshape(NS, -1, K)

    x4 = jnp.pad(jnp.concatenate(
        [node_feats, jnp.ones((N, 1), jnp.float32)], axis=1),
        ((0, NPAD - N), (0, 0)))
    x16 = jnp.pad(x4, ((0, 0), (0, 12)))
    zeros16 = jnp.zeros((NPAD, 16), jnp.float32)
    ws1p = jnp.pad(Ws1, ((0, 1), (0, 0)))
    wn1p = jnp.pad(Wn1, ((0, 1), (0, 0)))

    agg1 = _sc_deg_agg1(x16, srca, dsta, zeros16)

    h1c, dinv = pl.pallas_call(
        _tc_h1_body,
        grid=(_NG,),
        in_specs=[
            pl.BlockSpec((_NB, 4), lambda i: (i, 0)),
            pl.BlockSpec((NC, _NB, 16), lambda i: (0, i, 0)),
            _full((4, 64)), _full((4, 64)), _full((1, 64)),
        ],
        out_specs=[
            pl.BlockSpec((4, _NB, 16), lambda i: (0, i, 0)),
            pl.BlockSpec((_NB, 1), lambda i: (i, 0)),
        ],
        out_shape=[
            jax.ShapeDtypeStruct((4, NPAD, 16), jnp.float32),
            jax.ShapeDtypeStruct((NPAD, 1), jnp.float32),
        ],
    )(x4, agg1, ws1p, wn1p, b1.reshape(1, 64))

    agg2 = _sc_agg2(h1c.reshape(4 * NPAD, 16), srcb, dstb, zeros16)

    h2 = pl.pallas_call(
        _tc_h2_body,
        grid=(_NG,),
        in_specs=[
            pl.BlockSpec((4, _NB, 16), lambda i: (0, i, 0)),
            pl.BlockSpec((2, NC, _NB, 16), lambda i: (0, 0, i, 0)),
            pl.BlockSpec((_NB, 1), lambda i: (i, 0)),
            _full((64, 64)), _full((64, 64)), _full((1, 64)),
        ],
        out_specs=pl.BlockSpec((_NB, 64), lambda i: (i, 0)),
        out_shape=jax.ShapeDtypeStruct((NPAD, 64), jnp.bfloat16),
    )(h1c, agg2, dinv, Ws2, Wn2, b2.reshape(1, 64))

    gpair = _sc_gather_pair(h2, srca, dsta).reshape(EPAD, 128)

    logits = pl.pallas_call(
        _tc_mlp_body,
        grid=(_EG,),
        in_specs=[
            pl.BlockSpec((_EB, 128), lambda i: (i, 0)),
            pl.BlockSpec((_EB, 4), lambda i: (i, 0)),
            _full((64, 128)), _full((64, 128)), _full((4, 128)),
            _full((1, 128)), _full((128, 64)), _full((1, 64)),
            _full((64, 1)), _full((1, 1)),
        ],
        out_specs=pl.BlockSpec((_EB, 1), lambda i: (i, 0)),
        out_shape=jax.ShapeDtypeStruct((E, 1), jnp.float32),
    )(gpair, edge_feats, We1[:64], We1[64:128], We1[128:],
      be1.reshape(1, 128), We2, be2.reshape(1, 64), We3, be3.reshape(1, 1))

    return logits
